# baseline (speedup 1.0000x reference)
import functools

import jax
import jax.numpy as jnp
from jax import lax
from jax.experimental import pallas as pl
from jax.experimental.pallas import tpu as pltpu
from jax.experimental.pallas import tpu_sc as plsc

B, N, S, D = 4096, 10, 20, 32
V = 1000000
R = B * S
NUM_CORES = 2
NUM_SUBCORES = 16
NW = NUM_CORES * NUM_SUBCORES
BPW = B // NW
G = 8
C = G * S
NCHUNK = BPW // G
LANES = 16

WROWS = V * D // 128
CW = 128
TFULL = WROWS // CW
TTAIL = WROWS - TFULL * CW


TPITCH = 520


def _transpose_chunk(tin, wout, rows):
    iota = lax.iota(jnp.int32, LANES)
    lo = iota
    hi = iota + LANES

    def row_body(r, carry):
        base = 4 * r
        for j in range(4):
            idx_i = jnp.broadcast_to(base + j, (LANES,))
            v0 = plsc.load_gather(tin, [lo, idx_i])
            v1 = plsc.load_gather(tin, [hi, idx_i])
            wout[r, pl.ds(j * D, LANES)] = v0
            wout[r, pl.ds(j * D + LANES, LANES)] = v1
        return carry

    lax.fori_loop(0, rows, row_body, 0, unroll=4)


def _detile_body(tt_hbm, tail_hbm, w_hbm, tin, wout, tail_v, sem):
    wid = lax.axis_index("s") * NUM_CORES + lax.axis_index("c")

    def chunk_body(k, carry):
        cb = wid + NW * k
        pltpu.sync_copy(tt_hbm.at[:, pl.ds(cb * 4 * CW, 4 * CW)],
                        tin.at[:, pl.ds(0, 4 * CW)])
        _transpose_chunk(tin, wout, CW)
        pltpu.sync_copy(wout, w_hbm.at[pl.ds(cb * CW, CW)])
        return carry

    nk = jnp.where(wid < TFULL % NW, TFULL // NW + 1, TFULL // NW)
    lax.fori_loop(0, nk, chunk_body, 0)

    @pl.when(wid == NW - 1)
    def _tail():
        pltpu.sync_copy(tail_hbm, tail_v)
        pltpu.sync_copy(tail_v, w_hbm.at[pl.ds(TFULL * CW, TTAIL)])


def _cbow_body(idx_hbm, table_hbm, out_hbm, idx_v, acc_v, sem):
    wid = lax.axis_index("s") * NUM_CORES + lax.axis_index("c")
    bbase = wid * BPW

    def chunk_body(ci, carry):
        b0 = bbase + ci * G
        pltpu.sync_copy(idx_hbm.at[pl.ds(b0, G)], idx_v)
        first = [
            pltpu.async_copy(
                table_hbm.at[idx_v.at[g, 0]], acc_v.at[pl.ds(g * S, S)], sem)
            for g in range(G)
        ]
        for cp in first:
            cp.wait()
        rest = [
            pltpu.async_copy(
                table_hbm.at[idx_v.at[g, n]], acc_v.at[pl.ds(g * S, S)], sem,
                add=True)
            for g in range(G)
            for n in range(1, N)
        ]
        for cp in rest:
            cp.wait()

        def row_body(r, c2):
            for h in range(0, D, LANES):
                acc_v[r, pl.ds(h, LANES)] = acc_v[r, pl.ds(h, LANES)] * 0.1
            return c2

        lax.fori_loop(0, C, row_body, 0, unroll=4)
        pltpu.sync_copy(acc_v, out_hbm.at[pl.ds(b0 * S, C)])
        return carry

    lax.fori_loop(0, NCHUNK, chunk_body, 0)


@jax.jit
def kernel(x, table):
    mesh = plsc.VectorSubcoreMesh(core_axis_name="c", subcore_axis_name="s")

    detile = pl.kernel(
        _detile_body,
        mesh=mesh,
        out_type=jax.ShapeDtypeStruct((WROWS, 128), jnp.float32),
        scratch_types=[
            pltpu.VMEM((D, TPITCH), jnp.float32),
            pltpu.VMEM((CW, 128), jnp.float32),
            pltpu.VMEM((TTAIL, 128), jnp.float32),
            pltpu.SemaphoreType.DMA,
        ],
        compiler_params=pltpu.CompilerParams(
            use_tc_tiling_on_sc=True, needs_layout_passes=False),
    )
    tail = table[TFULL * 4 * CW:].reshape(TTAIL, 128)
    w = detile(table.T, tail)
    table_lin = w.reshape(V, D)

    run = pl.kernel(
        _cbow_body,
        mesh=mesh,
        out_type=jax.ShapeDtypeStruct((R, D), jnp.float32),
        scratch_types=[
            pltpu.VMEM((G, N, S), jnp.int32),
            pltpu.VMEM((C, D), jnp.float32),
            pltpu.SemaphoreType.DMA,
        ],
        compiler_params=pltpu.CompilerParams(use_tc_tiling_on_sc=False),
    )
    out = run(x.astype(jnp.int32), table_lin)
    return out.reshape(B, 1, S, D)


# speedup vs baseline: 1.4529x; 1.4529x over previous
import functools

import jax
import jax.numpy as jnp
from jax import lax
from jax.experimental import pallas as pl
from jax.experimental.pallas import tpu as pltpu
from jax.experimental.pallas import tpu_sc as plsc

B, N, S, D = 4096, 10, 20, 32
VOCAB_ROWS = 1000000
R = B * S
NUM_CORES = 2
NUM_SUBCORES = 16
NW = NUM_CORES * NUM_SUBCORES
BPW = B // NW
G = 8
C = G * S
NCHUNK = BPW // G
LANES = 16


def _cbow_body(idx_hbm, table_hbm, out_hbm, idx_v, acc_v, sem):
    wid = lax.axis_index("s") * NUM_CORES + lax.axis_index("c")
    bbase = wid * BPW

    def chunk_body(ci, carry):
        b0 = bbase + ci * G
        pltpu.sync_copy(idx_hbm.at[pl.ds(b0, G)], idx_v)
        first = [
            pltpu.async_copy(
                table_hbm.at[idx_v.at[g, 0]], acc_v.at[pl.ds(g * S, S)], sem)
            for g in range(G)
        ]
        for cp in first:
            cp.wait()
        rest = [
            pltpu.async_copy(
                table_hbm.at[idx_v.at[g, n]], acc_v.at[pl.ds(g * S, S)], sem,
                add=True)
            for g in range(G)
            for n in range(1, N)
        ]
        for cp in rest:
            cp.wait()

        def row_body(r, c2):
            for h in range(0, D, LANES):
                acc_v[r, pl.ds(h, LANES)] = acc_v[r, pl.ds(h, LANES)] * 0.1
            return c2

        lax.fori_loop(0, C, row_body, 0, unroll=4)
        pltpu.sync_copy(acc_v, out_hbm.at[pl.ds(b0 * S, C)])
        return carry

    lax.fori_loop(0, NCHUNK, chunk_body, 0)


@jax.jit
def kernel(x, table):
    mesh = plsc.VectorSubcoreMesh(core_axis_name="c", subcore_axis_name="s")
    run = pl.kernel(
        _cbow_body,
        mesh=mesh,
        out_type=jax.ShapeDtypeStruct((R, D), jnp.float32),
        scratch_types=[
            pltpu.VMEM((G, N, S), jnp.int32),
            pltpu.VMEM((C, D), jnp.float32),
            pltpu.SemaphoreType.DMA,
        ],
        compiler_params=pltpu.CompilerParams(use_tc_tiling_on_sc=False),
    )
    out = run(x.astype(jnp.int32), table)
    return out.reshape(B, 1, S, D)
